# baseline (speedup 1.0000x reference)
import functools

import jax
import jax.numpy as jnp
from jax import lax
from jax.experimental import pallas as pl
from jax.experimental.pallas import tpu as pltpu
from jax.experimental.pallas import tpu_sc as plsc


N_EMB = 5
N_INT = 5
GRP = 50
NG = 20
KW = 5 * GRP
GW = 256
CB = 512


def _sc_gather_params(gene_ix, W1r, b1, W2r, b2, n_idx_padded):
    n_cores = 2
    bpc = n_idx_padded // n_cores
    mesh = plsc.ScalarSubcoreMesh(axis_name="core", num_cores=n_cores)
    out_type = [
        jax.ShapeDtypeStruct((n_idx_padded, 25), jnp.float32),
        jax.ShapeDtypeStruct((n_idx_padded, 5), jnp.float32),
        jax.ShapeDtypeStruct((n_idx_padded, 5), jnp.float32),
        jax.ShapeDtypeStruct((n_idx_padded, 1), jnp.float32),
    ]

    @functools.partial(
        pl.kernel,
        mesh=mesh,
        out_type=out_type,
        scratch_types=[
            pltpu.SMEM((bpc,), jnp.int32),
            pltpu.SemaphoreType.DMA,
        ],
    )
    def gather_kernel(w1_hbm, b1_hbm, w2_hbm, b2_hbm, i_hbm, o1, o2, o3, o4,
                      idx_s, sem):
        base = lax.axis_index("core") * bpc
        pltpu.async_copy(i_hbm.at[pl.ds(base, bpc)], idx_s, sem).wait()
        win = 8

        @pl.loop(0, bpc + win)
        def _(k):
            @pl.when(k < bpc)
            def _issue():
                i = idx_s[k]
                pltpu.async_copy(w1_hbm.at[i], o1.at[base + k], sem)
                pltpu.async_copy(b1_hbm.at[i], o2.at[base + k], sem)
                pltpu.async_copy(w2_hbm.at[i], o3.at[base + k], sem)
                pltpu.async_copy(b2_hbm.at[i], o4.at[base + k], sem)

            @pl.when(k >= win)
            def _drain():
                j = k - win
                i2 = idx_s[j]
                pltpu.make_async_copy(w1_hbm.at[i2], o1.at[base + j], sem).wait()
                pltpu.make_async_copy(b1_hbm.at[i2], o2.at[base + j], sem).wait()
                pltpu.make_async_copy(w2_hbm.at[i2], o3.at[base + j], sem).wait()
                pltpu.make_async_copy(b2_hbm.at[i2], o4.at[base + j], sem).wait()

    idx = jnp.pad(gene_ix.astype(jnp.int32), (0, n_idx_padded - gene_ix.shape[0]))
    return gather_kernel(W1r, b1, W2r, b2, idx)


def _mlp_body(x_ref, vdt_ref, b1r_ref, b2r_ref, out_ref, w1bd_ref, w2bd_ref):

    @pl.when(pl.program_id(0) == 0)
    def _build_blockdiag():
        rowi = lax.broadcasted_iota(jnp.int32, (GW, GW), 0)
        coli = lax.broadcasted_iota(jnp.int32, (GW, GW), 1)
        diag_masks = [(coli - rowi) == d for d in range(-4, 5)]
        r2 = lax.broadcasted_iota(jnp.int32, (GW, 64), 0)
        c2 = lax.broadcasted_iota(jnp.int32, (GW, 64), 1)
        m2 = (r2 // 5 == c2) & (r2 < KW) & (c2 < GRP)
        for g in range(NG):
            base = vdt_ref[g * GW : (g + 1) * GW, :]
            acc = jnp.zeros((GW, GW), jnp.float32)
            for d in range(9):
                acc = acc + jnp.where(diag_masks[d], base[:, d : d + 1], 0.0)
            w1bd_ref[g] = acc.astype(jnp.bfloat16)
            w2 = jnp.where(m2, base[:, 9:10], 0.0)
            w2bd_ref[g] = w2.astype(jnp.bfloat16)

    xb = x_ref[...].astype(jnp.bfloat16)
    outs = []
    for g in range(NG):
        xg = xb[:, KW * g : KW * (g + 1)]
        h = lax.dot_general(
            xg, w1bd_ref[g][:KW, :],
            (((1,), (0,)), ((), ())),
            preferred_element_type=jnp.float32,
        )
        h = h + b1r_ref[0:1, GW * g : GW * (g + 1)]
        h = jax.nn.sigmoid(h).astype(jnp.bfloat16)
        o = lax.dot_general(
            h, w2bd_ref[g],
            (((1,), (0,)), ((), ())),
            preferred_element_type=jnp.float32,
        )
        outs.append(o[:, :GRP])
    out_ref[...] = jnp.concatenate(outs, axis=1) + b2r_ref[...]


def _dense_mlp(xr, vdt, b1r, b2r):
    C = xr.shape[0]
    return pl.pallas_call(
        _mlp_body,
        grid=(C // CB,),
        in_specs=[
            pl.BlockSpec((CB, 5 * NG * GRP), lambda c: (c, 0)),
            pl.BlockSpec((NG * GW, 16), lambda c: (0, 0)),
            pl.BlockSpec((1, NG * GW), lambda c: (0, 0)),
            pl.BlockSpec((1, NG * GRP), lambda c: (0, 0)),
        ],
        out_specs=pl.BlockSpec((CB, NG * GRP), lambda c: (c, 0)),
        out_shape=jax.ShapeDtypeStruct((C, NG * GRP), jnp.float32),
        scratch_shapes=[
            pltpu.VMEM((NG, GW, GW), jnp.bfloat16),
            pltpu.VMEM((NG, GW, 64), jnp.bfloat16),
        ],
    )(xr, vdt, b1r, b2r)


def _prep_params(w1g, b1g, w2g, b2g):
    G = w1g.shape[0]
    w1 = w1g.reshape(G, N_EMB, N_INT)
    w1p = jnp.pad(w1, ((0, 0), (0, 0), (4, 4)))
    i_idx = jnp.arange(N_EMB)[None, :, None]
    d_idx = jnp.arange(9)[None, None, :]
    ta = jnp.take_along_axis(w1p, jnp.broadcast_to(i_idx + d_idx, (G, 5, 9)), axis=2)
    v = ta.reshape(NG, KW, 9)
    vdt = jnp.concatenate(
        [v, w2g.reshape(NG, KW, 1), jnp.zeros((NG, KW, 6), jnp.float32)], axis=2
    )
    vdt = jnp.pad(vdt, ((0, 0), (0, GW - KW), (0, 0))).reshape(NG * GW, 16)
    b1r = jnp.pad(b1g.reshape(NG, KW), ((0, 0), (0, GW - KW))).reshape(1, NG * GW)
    b2r = b2g.reshape(1, NG * GRP)
    return vdt, b1r, b2r


def kernel(cell_gene_embedding, gene_ix, W1, b1, W2, b2):
    N = W1.shape[0]
    C = cell_gene_embedding.shape[0]
    G = gene_ix.shape[0]
    Gp = 1024

    w1g, b1g, w2g, b2g = _sc_gather_params(
        gene_ix, W1.reshape(N, 25), b1, W2.reshape(N, N_INT), b2, Gp
    )
    vdt, b1r, b2r = _prep_params(w1g[:G], b1g[:G], w2g[:G], b2g[:G])
    xr = cell_gene_embedding.reshape(C, G * N_EMB)
    return _dense_mlp(xr, vdt, b1r, b2r)
